# Initial kernel scaffold; baseline (speedup 1.0000x reference)
import jax
import jax.numpy as jnp
from jax.experimental import pallas as pl


def kernel(s4, s8, s16, s32, gate_w1, gate_b1, gate_w2, gate_b2, exp_w1, exp_b1, exp_w2, exp_b2):
    raise NotImplementedError("write your pallas kernel here")


# speedup vs baseline: 1.4200x; 1.4200x over previous
import jax
import jax.numpy as jnp
from jax.experimental import pallas as pl
from jax.experimental.pallas import tpu as pltpu

B = 4
E = 8
C = 512
T = B * 16 * 16
EPAD = 128


def _moe_dense_body(tok_ref, gw1_ref, gb1_ref, gw2_ref, gb2_ref,
                    w1_ref, b1_ref, w2_ref, b2_ref,
                    out_ref, idx_scr, p_scr):
    e = pl.program_id(0)

    @pl.when(e == 0)
    def _gate():
        g1 = jax.nn.gelu(
            jax.lax.dot_general(tok_ref[...], gw1_ref[...],
                                (((1,), (0,)), ((), ())),
                                preferred_element_type=jnp.float32)
            + gb1_ref[...])
        logits = jax.lax.dot_general(g1, gw2_ref[...],
                                     (((1,), (0,)), ((), ())),
                                     preferred_element_type=jnp.float32)
        logits = logits + gb2_ref[...]
        col = jax.lax.broadcasted_iota(jnp.int32, (T, EPAD), 1)
        logits = jnp.where(col < E, logits, -1e30)
        m = jnp.max(logits, axis=1, keepdims=True)
        ex = jnp.exp(logits - m)
        denom = jnp.sum(ex, axis=1, keepdims=True)
        p_scr[...] = 1.0 / denom
        hit = logits == m
        idx_scr[...] = jnp.min(jnp.where(hit, col, EPAD), axis=1, keepdims=True)
        out_ref[...] = tok_ref[...]

    h = jax.nn.gelu(
        jax.lax.dot_general(tok_ref[...], w1_ref[0],
                            (((1,), (0,)), ((), ())),
                            preferred_element_type=jnp.float32)
        + b1_ref[0])
    y = jax.lax.dot_general(h, w2_ref[0],
                            (((1,), (0,)), ((), ())),
                            preferred_element_type=jnp.float32) + b2_ref[0]
    scale = jnp.where(idx_scr[...] == e, p_scr[...], 0.0)
    out_ref[...] += scale * y


def kernel(s4, s8, s16, s32, gate_w1, gate_b1, gate_w2, gate_b2,
           exp_w1, exp_b1, exp_w2, exp_b2):
    tok = jnp.transpose(s32, (0, 2, 3, 1)).reshape(T, C)
    gw2p = jnp.zeros((C, EPAD), jnp.float32).at[:, :E].set(gate_w2)
    gb2p = jnp.zeros((1, EPAD), jnp.float32).at[0, :E].set(gate_b2)

    y_tok = pl.pallas_call(
        _moe_dense_body,
        grid=(E,),
        in_specs=[
            pl.BlockSpec((T, C), lambda e: (0, 0)),
            pl.BlockSpec((C, C), lambda e: (0, 0)),
            pl.BlockSpec((1, C), lambda e: (0, 0)),
            pl.BlockSpec((C, EPAD), lambda e: (0, 0)),
            pl.BlockSpec((1, EPAD), lambda e: (0, 0)),
            pl.BlockSpec((1, C, C), lambda e: (e, 0, 0)),
            pl.BlockSpec((1, 1, C), lambda e: (e, 0, 0)),
            pl.BlockSpec((1, C, C), lambda e: (e, 0, 0)),
            pl.BlockSpec((1, 1, C), lambda e: (e, 0, 0)),
        ],
        out_specs=pl.BlockSpec((T, C), lambda e: (0, 0)),
        out_shape=jax.ShapeDtypeStruct((T, C), jnp.float32),
        scratch_shapes=[
            pltpu.VMEM((T, 1), jnp.int32),
            pltpu.VMEM((T, 1), jnp.float32),
        ],
    )(tok, gate_w1, gate_b1.reshape(1, C), gw2p, gb2p,
      exp_w1, exp_b1.reshape(E, 1, C), exp_w2, exp_b2.reshape(E, 1, C))

    s32_out = jnp.transpose(y_tok.reshape(B, 16, 16, C), (0, 3, 1, 2))
    return (s4, s8, s16, s32_out)
